# Initial kernel scaffold; baseline (speedup 1.0000x reference)
import jax
import jax.numpy as jnp
from jax.experimental import pallas as pl


def kernel(x, edge_index):
    raise NotImplementedError("write your pallas kernel here")


# speedup vs baseline: 3.1349x; 3.1349x over previous
import functools
from math import comb

import jax
import jax.numpy as jnp
from jax import lax
from jax.experimental import pallas as pl
from jax.experimental.pallas import tpu as pltpu
from jax.experimental.pallas import tpu_sc as plsc

N = 10000
E = 320000
D = 128
K = 10

NC = 2
NS = 16
NW = NC * NS
CL = 128
TCH = 80
E_PAD = TCH * NW * CL
N_PAD = 10112
ZPT = N_PAD // NS

_mesh = plsc.VectorSubcoreMesh(core_axis_name="c", subcore_axis_name="s")


@functools.partial(
    pl.kernel,
    mesh=_mesh,
    out_type=jax.ShapeDtypeStruct((NC * N_PAD, D), jnp.float32),
    scratch_types=[
        pltpu.VMEM((TCH, CL), jnp.int32),
        pltpu.VMEM((TCH, CL), jnp.int32),
        pltpu.VMEM((CL, D), jnp.float32),
        pltpu.VMEM_SHARED((N_PAD, D), jnp.float32),
        pltpu.SemaphoreType.DMA,
    ],
)
def _sc_spmm(g_hbm, col_hbm, row_hbm, zeros_hbm, out_hbm,
             colv, rowv, buf, acc, sem):
    c = lax.axis_index("c")
    s = lax.axis_index("s")
    wid = s * NC + c
    pltpu.sync_copy(zeros_hbm.at[pl.ds(s * ZPT, ZPT)], acc.at[pl.ds(s * ZPT, ZPT)])
    pltpu.sync_copy(col_hbm.at[pl.ds(wid * TCH, TCH)], colv)
    pltpu.sync_copy(row_hbm.at[pl.ds(wid * TCH, TCH)], rowv)
    plsc.subcore_barrier()

    def body(j, carry):
        pltpu.async_copy(g_hbm.at[colv.at[j]], buf, sem).wait()
        pltpu.sync_copy(buf, acc.at[rowv.at[j]], add=True)
        return carry

    lax.fori_loop(0, TCH, body, 0)
    plsc.subcore_barrier()
    pltpu.sync_copy(acc.at[pl.ds(s * ZPT, ZPT)],
                    out_hbm.at[pl.ds(c * N_PAD + s * ZPT, ZPT)])


BR = 400
GRID = N // BR


def _tc_pre_body(x_ref, dp0_ref, dp1_ref, g0_ref, d2_ref, sqd_ref):
    deg = dp0_ref[...] + dp1_ref[...] + 1.0
    dinv = lax.rsqrt(deg)
    g0_ref[...] = x_ref[...] * dinv
    d2_ref[...] = 1.0 / deg
    sqd_ref[...] = deg * dinv


def _tc_precompute(x, dp0, dp1):
    return pl.pallas_call(
        _tc_pre_body,
        grid=(GRID,),
        in_specs=[
            pl.BlockSpec((BR, D), lambda i: (i, 0)),
            pl.BlockSpec((BR, 1), lambda i: (i, 0)),
            pl.BlockSpec((BR, 1), lambda i: (i, 0)),
        ],
        out_specs=[
            pl.BlockSpec((BR, D), lambda i: (i, 0)),
            pl.BlockSpec((BR, 1), lambda i: (i, 0)),
            pl.BlockSpec((BR, 1), lambda i: (i, 0)),
        ],
        out_shape=[
            jax.ShapeDtypeStruct((N, D), jnp.float32),
            jax.ShapeDtypeStruct((N, 1), jnp.float32),
            jax.ShapeDtypeStruct((N, 1), jnp.float32),
        ],
    )(x, dp0, dp1)


def _tc_update_body(g_ref, s0_ref, s1_ref, d2_ref, out_ref):
    g = g_ref[...]
    stot = s0_ref[...] + s1_ref[...] + g
    out_ref[...] = 0.5 * g - 0.5 * d2_ref[...] * stot


def _tc_update(g, s0, s1, d2):
    return pl.pallas_call(
        _tc_update_body,
        grid=(GRID,),
        in_specs=[
            pl.BlockSpec((BR, D), lambda i: (i, 0)),
            pl.BlockSpec((BR, D), lambda i: (i, 0)),
            pl.BlockSpec((BR, D), lambda i: (i, 0)),
            pl.BlockSpec((BR, 1), lambda i: (i, 0)),
        ],
        out_specs=pl.BlockSpec((BR, D), lambda i: (i, 0)),
        out_shape=jax.ShapeDtypeStruct((N, D), jnp.float32),
    )(g, s0, s1, d2)


CMAT = [[0.0] * (K + 1) for _ in range(K + 1)]
for k in range(K + 1):
    for j in range(K - k + 1):
        CMAT[k][k + j] = float(((-1) ** j) * comb(K, k) * comb(K - k, j))


def _tc_combine_body(sqd_ref, *refs):
    g_refs = refs[:K + 1]
    out_ref = refs[K + 1]
    sq = sqd_ref[...]
    gs = [r[...] for r in g_refs]
    for k in range(K + 1):
        acc = None
        for m in range(k, K + 1):
            term = CMAT[k][m] * gs[m]
            acc = term if acc is None else acc + term
        out_ref[k, :, :] = acc * sq


def _tc_combine(sqd, gs):
    in_specs = [pl.BlockSpec((BR, 1), lambda i: (i, 0))]
    in_specs += [pl.BlockSpec((BR, D), lambda i: (i, 0)) for _ in range(K + 1)]
    return pl.pallas_call(
        _tc_combine_body,
        grid=(GRID,),
        in_specs=in_specs,
        out_specs=pl.BlockSpec((K + 1, BR, D), lambda i: (0, i, 0)),
        out_shape=jax.ShapeDtypeStruct((K + 1, N, D), jnp.float32),
    )(sqd, *gs)


@jax.jit
def kernel(x, edge_index):
    row = edge_index[0].astype(jnp.int32)
    col = edge_index[1].astype(jnp.int32)
    pad = E_PAD - E
    row_p = jnp.concatenate([row, jnp.full((pad,), N, jnp.int32)])
    col_p = jnp.concatenate([col, jnp.zeros((pad,), jnp.int32)])
    row2 = row_p.reshape(NW * TCH, CL)
    col2 = col_p.reshape(NW * TCH, CL)

    zeros_d = jnp.zeros((N_PAD, D), jnp.float32)
    ones_nd = jnp.ones((N, D), jnp.float32)

    dsp = _sc_spmm(ones_nd, col2, row2, zeros_d)
    g0, d2, sqd = _tc_precompute(x, dsp[:N, 0:1], dsp[N_PAD:N_PAD + N, 0:1])

    gs = [g0]
    g = g0
    for _ in range(K):
        sparts = _sc_spmm(g, col2, row2, zeros_d)
        g = _tc_update(g, sparts[:N], sparts[N_PAD:N_PAD + N], d2)
        gs.append(g)

    return _tc_combine(sqd, gs)


# speedup vs baseline: 3.4508x; 1.1008x over previous
import functools
from math import comb

import jax
import jax.numpy as jnp
from jax import lax
from jax.experimental import pallas as pl
from jax.experimental.pallas import tpu as pltpu
from jax.experimental.pallas import tpu_sc as plsc

N = 10000
E = 320000
D = 128
K = 10

NC = 2
NS = 16
NW = NC * NS
CL = 128
TCH = 80
E_PAD = TCH * NW * CL
N_PAD = 10112
ZPT = N_PAD // NS
NBUF = 2
CPB = 16
NBLK = TCH // CPB

_mesh = plsc.VectorSubcoreMesh(core_axis_name="c", subcore_axis_name="s")


@functools.partial(
    pl.kernel,
    mesh=_mesh,
    out_type=jax.ShapeDtypeStruct((NC * N_PAD, D), jnp.float32),
    scratch_types=[
        pltpu.VMEM((CPB, CL), jnp.int32),
        pltpu.VMEM((CPB, CL), jnp.int32),
    ] + [pltpu.VMEM((CL, D), jnp.float32)] * NBUF + [
        pltpu.VMEM_SHARED((N_PAD, D), jnp.float32),
    ] + [pltpu.SemaphoreType.DMA] * NBUF,
)
def _sc_spmm(g_hbm, col_hbm, row_hbm, zeros_hbm, out_hbm,
             colv, rowv, *rest):
    bufs = rest[:NBUF]
    acc = rest[NBUF]
    sems = rest[NBUF + 1:]
    c = lax.axis_index("c")
    s = lax.axis_index("s")
    wid = s * NC + c
    pltpu.sync_copy(zeros_hbm.at[pl.ds(s * ZPT, ZPT)], acc.at[pl.ds(s * ZPT, ZPT)])
    plsc.subcore_barrier()

    def outer(bi, carry):
        base = wid * TCH + bi * CPB
        pltpu.sync_copy(col_hbm.at[pl.ds(base, CPB)], colv)
        pltpu.sync_copy(row_hbm.at[pl.ds(base, CPB)], rowv)
        for b in range(NBUF):
            pltpu.make_async_copy(g_hbm.at[colv.at[b]], bufs[b], sems[b]).start()

        def inner(g_i, carry2):
            for b in range(NBUF):
                j = NBUF * g_i + b
                pltpu.make_async_copy(g_hbm.at[colv.at[j]], bufs[b], sems[b]).wait()
                pltpu.sync_copy(bufs[b], acc.at[rowv.at[j]], add=True)
                jn = j + NBUF

                @pl.when(jn < CPB)
                def _():
                    pltpu.make_async_copy(
                        g_hbm.at[colv.at[jn]], bufs[b], sems[b]).start()
            return carry2

        lax.fori_loop(0, CPB // NBUF, inner, 0)
        return carry

    lax.fori_loop(0, NBLK, outer, 0)
    plsc.subcore_barrier()
    pltpu.sync_copy(acc.at[pl.ds(s * ZPT, ZPT)],
                    out_hbm.at[pl.ds(c * N_PAD + s * ZPT, ZPT)])


BR = 400
GRID = N // BR


def _tc_pre_body(x_ref, dp0_ref, dp1_ref, g0_ref, d2_ref, sqd_ref):
    deg = dp0_ref[...] + dp1_ref[...] + 1.0
    dinv = lax.rsqrt(deg)
    g0_ref[...] = x_ref[...] * dinv
    d2_ref[...] = 1.0 / deg
    sqd_ref[...] = deg * dinv


def _tc_precompute(x, dp0, dp1):
    return pl.pallas_call(
        _tc_pre_body,
        grid=(GRID,),
        in_specs=[
            pl.BlockSpec((BR, D), lambda i: (i, 0)),
            pl.BlockSpec((BR, 1), lambda i: (i, 0)),
            pl.BlockSpec((BR, 1), lambda i: (i, 0)),
        ],
        out_specs=[
            pl.BlockSpec((BR, D), lambda i: (i, 0)),
            pl.BlockSpec((BR, 1), lambda i: (i, 0)),
            pl.BlockSpec((BR, 1), lambda i: (i, 0)),
        ],
        out_shape=[
            jax.ShapeDtypeStruct((N, D), jnp.float32),
            jax.ShapeDtypeStruct((N, 1), jnp.float32),
            jax.ShapeDtypeStruct((N, 1), jnp.float32),
        ],
    )(x, dp0, dp1)


def _tc_update_body(g_ref, s0_ref, s1_ref, d2_ref, out_ref):
    g = g_ref[...]
    stot = s0_ref[...] + s1_ref[...] + g
    out_ref[...] = 0.5 * g - 0.5 * d2_ref[...] * stot


def _tc_update(g, s0, s1, d2):
    return pl.pallas_call(
        _tc_update_body,
        grid=(GRID,),
        in_specs=[
            pl.BlockSpec((BR, D), lambda i: (i, 0)),
            pl.BlockSpec((BR, D), lambda i: (i, 0)),
            pl.BlockSpec((BR, D), lambda i: (i, 0)),
            pl.BlockSpec((BR, 1), lambda i: (i, 0)),
        ],
        out_specs=pl.BlockSpec((BR, D), lambda i: (i, 0)),
        out_shape=jax.ShapeDtypeStruct((N, D), jnp.float32),
    )(g, s0, s1, d2)


CMAT = [[0.0] * (K + 1) for _ in range(K + 1)]
for k in range(K + 1):
    for j in range(K - k + 1):
        CMAT[k][k + j] = float(((-1) ** j) * comb(K, k) * comb(K - k, j))


def _tc_combine_body(sqd_ref, *refs):
    g_refs = refs[:K + 1]
    out_ref = refs[K + 1]
    sq = sqd_ref[...]
    gs = [r[...] for r in g_refs]
    for k in range(K + 1):
        acc = None
        for m in range(k, K + 1):
            term = CMAT[k][m] * gs[m]
            acc = term if acc is None else acc + term
        out_ref[k, :, :] = acc * sq


def _tc_combine(sqd, gs):
    in_specs = [pl.BlockSpec((BR, 1), lambda i: (i, 0))]
    in_specs += [pl.BlockSpec((BR, D), lambda i: (i, 0)) for _ in range(K + 1)]
    return pl.pallas_call(
        _tc_combine_body,
        grid=(GRID,),
        in_specs=in_specs,
        out_specs=pl.BlockSpec((K + 1, BR, D), lambda i: (0, i, 0)),
        out_shape=jax.ShapeDtypeStruct((K + 1, N, D), jnp.float32),
    )(sqd, *gs)


@jax.jit
def kernel(x, edge_index):
    row = edge_index[0].astype(jnp.int32)
    col = edge_index[1].astype(jnp.int32)
    pad = E_PAD - E
    row_p = jnp.concatenate([row, jnp.full((pad,), N, jnp.int32)])
    col_p = jnp.concatenate([col, jnp.zeros((pad,), jnp.int32)])
    row2 = row_p.reshape(NW * TCH, CL)
    col2 = col_p.reshape(NW * TCH, CL)

    zeros_d = jnp.zeros((N_PAD, D), jnp.float32)
    ones_nd = jnp.ones((N, D), jnp.float32)

    dsp = _sc_spmm(ones_nd, col2, row2, zeros_d)
    g0, d2, sqd = _tc_precompute(x, dsp[:N, 0:1], dsp[N_PAD:N_PAD + N, 0:1])

    gs = [g0]
    g = g0
    for _ in range(K):
        sparts = _sc_spmm(g, col2, row2, zeros_d)
        g = _tc_update(g, sparts[:N], sparts[N_PAD:N_PAD + N], d2)
        gs.append(g)

    return _tc_combine(sqd, gs)
